# Initial kernel scaffold; baseline (speedup 1.0000x reference)
import jax
import jax.numpy as jnp
from jax.experimental import pallas as pl


def kernel(x, gate_w, gate_b, w1, b1, w2, b2):
    raise NotImplementedError("write your pallas kernel here")


# speedup vs baseline: 3.0289x; 3.0289x over previous
import functools

import jax
import jax.numpy as jnp
from jax.experimental import pallas as pl
from jax.experimental.pallas import tpu as pltpu

E = 8
TOP_K = 2
D = 1024
DFF = 4096
T = 2048
FB = 1024
F = DFF // FB
_INV_SQRT2 = 0.7071067811865476


def _gelu(v):
    return 0.5 * v * (1.0 + jax.lax.erf(v * _INV_SQRT2))


def _router_body(x_ref, gw_ref, gb_ref, logits_ref, wd_ref):
    h = x_ref[...]
    logits = jnp.dot(h, gw_ref[...], preferred_element_type=jnp.float32)
    logits = logits + gb_ref[...]
    logits_ref[...] = logits
    e_iota = jax.lax.broadcasted_iota(jnp.int32, (T, E), 1)
    m0 = jnp.max(logits, axis=1, keepdims=True)
    s0 = jnp.min(jnp.where(logits == m0, e_iota, E), axis=1, keepdims=True)
    masked = jnp.where(e_iota == s0, -jnp.inf, logits)
    m1 = jnp.max(masked, axis=1, keepdims=True)
    s1 = jnp.min(jnp.where(masked == m1, e_iota, E), axis=1, keepdims=True)
    t = jnp.exp(m1 - m0)
    w0 = 1.0 / (1.0 + t)
    w1 = 1.0 - w0
    wd = jnp.where(e_iota == s0, w0, 0.0) + jnp.where(e_iota == s1, w1, 0.0)
    wd_ref[...] = wd


def _router(x2d, gate_w, gate_b):
    return pl.pallas_call(
        _router_body,
        out_shape=(
            jax.ShapeDtypeStruct((T, E), jnp.float32),
            jax.ShapeDtypeStruct((T, E), jnp.float32),
        ),
    )(x2d, gate_w, gate_b.reshape(1, E))


def _ffn_body(x_ref, wd_ref, w1_ref, b1_ref, w2_ref, b2_ref, out_ref, acc_ref):
    e = pl.program_id(0)
    f = pl.program_id(1)
    h1 = jnp.dot(x_ref[...], w1_ref[0], preferred_element_type=jnp.float32)
    h1 = _gelu(h1 + b1_ref[0])
    part = jnp.dot(h1, w2_ref[0], preferred_element_type=jnp.float32)
    e_iota = jax.lax.broadcasted_iota(jnp.int32, (T, E), 1)
    wcol = jnp.sum(jnp.where(e_iota == e, wd_ref[...], 0.0), axis=1,
                   keepdims=True)
    inc = part + jnp.where(f == 0, 1.0, 0.0) * b2_ref[0]
    inc = inc * wcol

    @pl.when((e == 0) & (f == 0))
    def _():
        acc_ref[...] = inc

    @pl.when(~((e == 0) & (f == 0)))
    def _():
        acc_ref[...] = acc_ref[...] + inc

    @pl.when((e == E - 1) & (f == F - 1))
    def _():
        out_ref[...] = acc_ref[...]


def _ffn(x2d, wd, w1, b1, w2, b2):
    return pl.pallas_call(
        _ffn_body,
        grid=(E, F),
        in_specs=[
            pl.BlockSpec((T, D), lambda e, f: (0, 0)),
            pl.BlockSpec((T, E), lambda e, f: (0, 0)),
            pl.BlockSpec((1, D, FB), lambda e, f: (e, 0, f)),
            pl.BlockSpec((1, 1, FB), lambda e, f: (e, 0, f)),
            pl.BlockSpec((1, FB, D), lambda e, f: (e, f, 0)),
            pl.BlockSpec((1, 1, D), lambda e, f: (e, 0, 0)),
        ],
        out_specs=pl.BlockSpec((T, D), lambda e, f: (0, 0)),
        out_shape=jax.ShapeDtypeStruct((T, D), jnp.float32),
        scratch_shapes=[pltpu.VMEM((T, D), jnp.float32)],
        compiler_params=pltpu.CompilerParams(
            dimension_semantics=("arbitrary", "arbitrary")),
    )(x2d, wd, w1, b1.reshape(E, 1, DFF), w2, b2.reshape(E, 1, D))


@jax.jit
def kernel(x, gate_w, gate_b, w1, b1, w2, b2):
    bsz, seq, dim = x.shape
    h = x.reshape(-1, dim)
    logits, wd = _router(h, gate_w, gate_b)
    final = _ffn(h, wd, w1, b1, w2, b2)
    return final.reshape(bsz, seq, dim), logits
